# Initial kernel scaffold; baseline (speedup 1.0000x reference)
import jax
import jax.numpy as jnp
from jax.experimental import pallas as pl


def kernel(x, edge_index, edge_attr, params):
    raise NotImplementedError("write your pallas kernel here")


# speedup vs baseline: 2.0227x; 2.0227x over previous
import functools

import jax
import jax.numpy as jnp
from jax import lax
from jax.experimental import pallas as pl
from jax.experimental.pallas import tpu as pltpu
from jax.experimental.pallas import tpu_sc as plsc

F32 = jnp.float32
I32 = jnp.int32
_H = 128
_CH = 128
_NT = 16
_NW = 32

_NPAD = 10240
_BK = _NPAD // _NW
_EPAD = 321536
_PCH = 1024
_NCHP = _EPAD // _PCH
_CAP = 11904


def _full(shape):
    return pl.BlockSpec(shape, lambda i: tuple(0 for _ in shape))


def _mlp_body(x_ref, w1_ref, b1_ref, w2_ref, b2_ref, o_ref):
    h = jnp.dot(x_ref[...], w1_ref[...], preferred_element_type=F32) + b1_ref[...]
    h = jnp.maximum(h, 0.0)
    o_ref[...] = jnp.dot(h, w2_ref[...], preferred_element_type=F32) + b2_ref[...]


def _tc_mlp(x, w1, b1, w2, b2, br):
    r, din = x.shape
    dh = w1.shape[1]
    dout = w2.shape[1]
    return pl.pallas_call(
        _mlp_body,
        grid=(r // br,),
        in_specs=[
            pl.BlockSpec((br, din), lambda i: (i, 0)),
            _full((din, dh)),
            _full((1, dh)),
            _full((dh, dout)),
            _full((1, dout)),
        ],
        out_specs=pl.BlockSpec((br, dout), lambda i: (i, 0)),
        out_shape=jax.ShapeDtypeStruct((r, dout), F32),
    )(x, w1, b1, w2, b2)


def _lin_body(x_ref, w_ref, b_ref, o_ref):
    o_ref[...] = jnp.dot(x_ref[...], w_ref[...], preferred_element_type=F32) + b_ref[...]


def _tc_lin(x, w, b, br):
    r, din = x.shape
    dout = w.shape[1]
    return pl.pallas_call(
        _lin_body,
        grid=(r // br,),
        in_specs=[
            pl.BlockSpec((br, din), lambda i: (i, 0)),
            _full((din, dout)),
            _full((1, dout)),
        ],
        out_specs=pl.BlockSpec((br, dout), lambda i: (i, 0)),
        out_shape=jax.ShapeDtypeStruct((r, dout), F32),
    )(x, w, b)


def _pre_body(h_ref, wa_ref, wb_ref, g_ref, f_ref):
    hb = h_ref[...]
    g_ref[...] = jnp.dot(hb, wa_ref[...], preferred_element_type=F32)
    f_ref[...] = jnp.dot(hb, wb_ref[...], preferred_element_type=F32)


def _tc_pre(h, wa, wb, br):
    r, d = h.shape
    return pl.pallas_call(
        _pre_body,
        grid=(r // br,),
        in_specs=[
            pl.BlockSpec((br, d), lambda i: (i, 0)),
            _full((d, _H)),
            _full((d, _H)),
        ],
        out_specs=[
            pl.BlockSpec((br, _H), lambda i: (i, 0)),
            pl.BlockSpec((br, _H), lambda i: (i, 0)),
        ],
        out_shape=[
            jax.ShapeDtypeStruct((r, _H), F32),
            jax.ShapeDtypeStruct((r, _H), F32),
        ],
    )(h, wa, wb)


def _post_body(h_ref, s_ref, w2_ref, nwa_ref, nwb_ref,
               nb1_ref, nw2_ref, nb2_ref, lg_ref, lb_ref, o_ref):
    agg = jnp.dot(s_ref[...], w2_ref[...], preferred_element_type=F32)
    h = h_ref[...]
    hh = (jnp.dot(h, nwa_ref[...], preferred_element_type=F32)
          + jnp.dot(agg, nwb_ref[...], preferred_element_type=F32) + nb1_ref[...])
    hh = jnp.maximum(hh, 0.0)
    t = h + jnp.dot(hh, nw2_ref[...], preferred_element_type=F32) + nb2_ref[...]
    m = jnp.mean(t, axis=-1, keepdims=True)
    v = jnp.mean((t - m) ** 2, axis=-1, keepdims=True)
    o_ref[...] = (t - m) * lax.rsqrt(v + 1e-5) * lg_ref[...] + lb_ref[...]


def _tc_post(h, s, w2, nwa, nwb, nb1, nw2, nb2, lg, lb, br):
    r, d = h.shape
    return pl.pallas_call(
        _post_body,
        grid=(r // br,),
        in_specs=[
            pl.BlockSpec((br, d), lambda i: (i, 0)),
            pl.BlockSpec((br, d), lambda i: (i, 0)),
            _full((d, d)),
            _full((d, d)),
            _full((d, d)),
            _full((1, d)),
            _full((d, d)),
            _full((1, d)),
            _full((1, d)),
            _full((1, d)),
        ],
        out_specs=pl.BlockSpec((br, d), lambda i: (i, 0)),
        out_shape=jax.ShapeDtypeStruct((r, d), F32),
    )(h, s, w2, nwa, nwb, nb1, nw2, nb2, lg, lb)


def _partition_sc(src, dst):
    mesh = plsc.VectorSubcoreMesh(core_axis_name="c", subcore_axis_name="s")

    @functools.partial(
        pl.kernel,
        out_type=(
            jax.ShapeDtypeStruct((_NW, 8, _CAP), I32),
            jax.ShapeDtypeStruct((_NW, 8, _CAP), I32),
            jax.ShapeDtypeStruct((_NW, 8, _CAP), I32),
            jax.ShapeDtypeStruct((_NW, 8, 16), I32),
        ),
        mesh=mesh,
        compiler_params=pltpu.CompilerParams(needs_layout_passes=False),
        scratch_types=[
            pltpu.VMEM((_PCH,), I32),
            pltpu.VMEM((_PCH,), I32),
            pltpu.VMEM((_CAP,), I32),
            pltpu.VMEM((_CAP,), I32),
            pltpu.VMEM((_CAP,), I32),
            pltpu.VMEM((16,), I32),
        ],
    )
    def body(src_hbm, dst_hbm, osrc, odstl, oeid, ocnt,
             srcv, dstv, bsrc, bdstl, beid, cntb):
        cid = lax.axis_index("c")
        sid = lax.axis_index("s")
        w = cid * _NT + sid
        zi = jnp.zeros((16,), I32)
        dumv = jnp.full((16,), _BK, I32)

        def pre(r, carry):
            sl = pl.ds(r * 16, 16)
            bsrc[sl] = zi
            beid[sl] = zi
            bdstl[sl] = dumv
            return carry

        lax.fori_loop(0, _CAP // 16, pre, 0)
        lo = w * _BK
        lanes = lax.iota(I32, 16)
        limit = _CAP - 16

        def chunk(i, off):
            base = i * _PCH
            pltpu.sync_copy(src_hbm.at[pl.ds(base, _PCH)], srcv)
            pltpu.sync_copy(dst_hbm.at[pl.ds(base, _PCH)], dstv)

            def slc(j, off2):
                sl = pl.ds(j * 16, 16)
                sv = srcv[sl]
                dv = dstv[sl]
                eid = base + j * 16 + lanes
                msk = (dv >= lo) & (dv < lo + _BK)
                pos = plsc.cumsum(msk.astype(I32))
                idx = jnp.minimum(off2 + pos - 1, limit)
                plsc.store_scatter(bsrc, [idx], sv, mask=msk)
                plsc.store_scatter(bdstl, [idx], dv - lo, mask=msk)
                plsc.store_scatter(beid, [idx], eid, mask=msk)
                return jnp.minimum(off2 + jnp.max(pos), limit)

            return lax.fori_loop(0, _PCH // 16, slc, off)

        off = lax.fori_loop(0, _NCHP, chunk, 0)
        nch = (off + _CH - 1) // _CH
        cntb[...] = zi + nch
        pltpu.sync_copy(bsrc, osrc.at[w, 0])
        pltpu.sync_copy(bdstl, odstl.at[w, 0])
        pltpu.sync_copy(beid, oeid.at[w, 0])
        pltpu.sync_copy(cntb, ocnt.at[w, 0])

    return body(src, dst)


def _edge_sc(g, f, c, psrc, pdstl, peid, pcnt):
    mesh = plsc.VectorSubcoreMesh(core_axis_name="c", subcore_axis_name="s")

    @functools.partial(
        pl.kernel,
        out_type=jax.ShapeDtypeStruct((_NPAD, _H), F32),
        mesh=mesh,
        compiler_params=pltpu.CompilerParams(needs_layout_passes=False),
        scratch_types=[
            pltpu.VMEM((16,), I32),
            pltpu.VMEM((_CH,), I32),
            pltpu.VMEM((_CH,), I32),
            pltpu.VMEM((_CH + 16,), I32),
            pltpu.VMEM((_CH,), I32),
            pltpu.VMEM((_CH, _H), F32),
            pltpu.VMEM((_CH, _H), F32),
            pltpu.VMEM((_CH, _H), F32),
            pltpu.VMEM((_BK + 8, _H), F32),
            pltpu.SemaphoreType.DMA,
            pltpu.SemaphoreType.DMA,
            pltpu.SemaphoreType.DMA,
        ],
    )
    def body(g_hbm, f_hbm, c_hbm, psrc_hbm, pdstl_hbm, peid_hbm,
             pcnt_hbm, s_out,
             cntv, srcv, gidv, dstlv, eidv, gv, fv, cv, acc,
             sem0, sem1, sem2):
        cid = lax.axis_index("c")
        sid = lax.axis_index("s")
        w = cid * _NT + sid
        zv = jnp.zeros((16,), F32)

        def initr(r, carry):
            for j in range(_H // 16):
                acc[r, pl.ds(j * 16, 16)] = zv
            return carry

        lax.fori_loop(0, _BK + 8, initr, 0)
        row0 = w * _BK
        pltpu.sync_copy(pcnt_hbm.at[w, 0], cntv)
        nch = cntv[pl.ds(0, 16)][0]

        def chunk(i, carry):
            base = i * _CH
            pltpu.sync_copy(psrc_hbm.at[w, 0, pl.ds(base, _CH)], srcv)
            pltpu.sync_copy(pdstl_hbm.at[w, 0, pl.ds(base, _CH)],
                            dstlv.at[pl.ds(0, _CH)])
            pltpu.sync_copy(peid_hbm.at[w, 0, pl.ds(base, _CH)], eidv)

            def gidr(j, carry2):
                sl = pl.ds(j * 16, 16)
                gidv[sl] = jnp.minimum(dstlv[sl] + row0, _NPAD - 1)
                return carry2

            lax.fori_loop(0, _CH // 16, gidr, 0)
            cpg = pltpu.async_copy(g_hbm.at[srcv], gv, sem0)
            cpf = pltpu.async_copy(f_hbm.at[gidv], fv, sem1)
            cpc = pltpu.async_copy(c_hbm.at[eidv], cv, sem2)
            cpg.wait()
            cpf.wait()
            cpc.wait()

            def rowb(r, rc):
                row = dstlv[pl.ds(r, 16)][0]
                for j in range(_H // 16):
                    sl = pl.ds(j * 16, 16)
                    hid = jnp.maximum(gv[r, sl] + fv[r, sl] + cv[r, sl], 0.0)
                    plsc.addupdate(acc.at[row, sl], hid)
                return rc

            lax.fori_loop(0, _CH, rowb, 0)
            return carry

        lax.fori_loop(0, nch, chunk, 0)
        for r0, nr in ((0, 160), (160, 160)):
            pltpu.sync_copy(acc.at[pl.ds(r0, nr)], s_out.at[pl.ds(row0 + r0, nr)])

    return body(g, f, c, psrc, pdstl, peid, pcnt)


def kernel(x, edge_index, edge_attr, params):
    n, d = x.shape
    e = edge_index.shape[1]

    src = edge_index[0]
    dst = edge_index[1]
    src_p = jnp.concatenate([src, jnp.zeros((_EPAD - e,), I32)])
    dst_p = jnp.concatenate([dst, jnp.full((_EPAD - e,), n, I32)])
    ea_p = jnp.concatenate([edge_attr, jnp.zeros((_EPAD - e, edge_attr.shape[1]), F32)])
    x_p = jnp.concatenate([x, jnp.zeros((_NPAD - n, d), F32)])

    def r2(b):
        return b.reshape(1, -1)

    enc = params['enc']
    h = _tc_mlp(x_p, enc['W1'], r2(enc['b1']), enc['W2'], r2(enc['b2']), br=2048)
    psrc, pdstl, peid, pcnt = _partition_sc(src_p, dst_p)

    for lp in params['layers']:
        ew1 = lp['edge']['W1']
        w1a, w1b, w1c = ew1[:_H], ew1[_H:2 * _H], ew1[2 * _H:]
        c = _tc_lin(ea_p, w1c, r2(lp['edge']['b1']), br=2048)
        g, f = _tc_pre(h, w1a, w1b, br=2048)
        s = _edge_sc(g, f, c, psrc, pdstl, peid, pcnt)
        nw1 = lp['node']['W1']
        h = _tc_post(h, s, lp['edge']['W2'],
                     nw1[:_H], nw1[_H:], r2(lp['node']['b1']),
                     lp['node']['W2'], r2(lp['node']['b2']),
                     r2(lp['ln']['g']), r2(lp['ln']['b']), br=2048)

    hd = params['head']
    hw2 = jnp.zeros((_H, _H), F32).at[:, :1].set(hd['W2'])
    hb2 = jnp.zeros((1, _H), F32).at[0, 0].set(hd['b2'][0])
    out = _tc_mlp(h, hd['W1'], r2(hd['b1']), hw2, hb2, br=2048)
    return out[:n, :1]


# speedup vs baseline: 2.8228x; 1.3955x over previous
import functools

import jax
import jax.numpy as jnp
from jax import lax
from jax.experimental import pallas as pl
from jax.experimental.pallas import tpu as pltpu
from jax.experimental.pallas import tpu_sc as plsc

F32 = jnp.float32
I32 = jnp.int32
_H = 128
_CH = 128
_NT = 16
_NW = 32

_NPAD = 10240
_BK = _NPAD // _NW
_EPAD = 321536
_PCH = 1024
_NCHP = _EPAD // _PCH
_CAP = 11904


def _full(shape):
    return pl.BlockSpec(shape, lambda i: tuple(0 for _ in shape))


def _mlp_body(x_ref, w1_ref, b1_ref, w2_ref, b2_ref, o_ref):
    h = jnp.dot(x_ref[...], w1_ref[...], preferred_element_type=F32) + b1_ref[...]
    h = jnp.maximum(h, 0.0)
    o_ref[...] = jnp.dot(h, w2_ref[...], preferred_element_type=F32) + b2_ref[...]


def _tc_mlp(x, w1, b1, w2, b2, br):
    r, din = x.shape
    dh = w1.shape[1]
    dout = w2.shape[1]
    return pl.pallas_call(
        _mlp_body,
        grid=(r // br,),
        in_specs=[
            pl.BlockSpec((br, din), lambda i: (i, 0)),
            _full((din, dh)),
            _full((1, dh)),
            _full((dh, dout)),
            _full((1, dout)),
        ],
        out_specs=pl.BlockSpec((br, dout), lambda i: (i, 0)),
        out_shape=jax.ShapeDtypeStruct((r, dout), F32),
    )(x, w1, b1, w2, b2)


def _lin_body(x_ref, w_ref, b_ref, o_ref):
    o_ref[...] = jnp.dot(x_ref[...], w_ref[...], preferred_element_type=F32) + b_ref[...]


def _tc_lin(x, w, b, br):
    r, din = x.shape
    dout = w.shape[1]
    return pl.pallas_call(
        _lin_body,
        grid=(r // br,),
        in_specs=[
            pl.BlockSpec((br, din), lambda i: (i, 0)),
            _full((din, dout)),
            _full((1, dout)),
        ],
        out_specs=pl.BlockSpec((br, dout), lambda i: (i, 0)),
        out_shape=jax.ShapeDtypeStruct((r, dout), F32),
    )(x, w, b)


def _pre_body(h_ref, wa_ref, wb_ref, g_ref, f_ref):
    hb = h_ref[...]
    g_ref[...] = jnp.dot(hb, wa_ref[...], preferred_element_type=F32)
    f_ref[...] = jnp.dot(hb, wb_ref[...], preferred_element_type=F32)


def _tc_pre(h, wa, wb, br):
    r, d = h.shape
    return pl.pallas_call(
        _pre_body,
        grid=(r // br,),
        in_specs=[
            pl.BlockSpec((br, d), lambda i: (i, 0)),
            _full((d, _H)),
            _full((d, _H)),
        ],
        out_specs=[
            pl.BlockSpec((br, _H), lambda i: (i, 0)),
            pl.BlockSpec((br, _H), lambda i: (i, 0)),
        ],
        out_shape=[
            jax.ShapeDtypeStruct((r, _H), F32),
            jax.ShapeDtypeStruct((r, _H), F32),
        ],
    )(h, wa, wb)


def _post_body(h_ref, s_ref, w2_ref, nwa_ref, nwb_ref,
               nb1_ref, nw2_ref, nb2_ref, lg_ref, lb_ref, o_ref):
    agg = jnp.dot(s_ref[...], w2_ref[...], preferred_element_type=F32)
    h = h_ref[...]
    hh = (jnp.dot(h, nwa_ref[...], preferred_element_type=F32)
          + jnp.dot(agg, nwb_ref[...], preferred_element_type=F32) + nb1_ref[...])
    hh = jnp.maximum(hh, 0.0)
    t = h + jnp.dot(hh, nw2_ref[...], preferred_element_type=F32) + nb2_ref[...]
    m = jnp.mean(t, axis=-1, keepdims=True)
    v = jnp.mean((t - m) ** 2, axis=-1, keepdims=True)
    o_ref[...] = (t - m) * lax.rsqrt(v + 1e-5) * lg_ref[...] + lb_ref[...]


def _tc_post(h, s, w2, nwa, nwb, nb1, nw2, nb2, lg, lb, br):
    r, d = h.shape
    return pl.pallas_call(
        _post_body,
        grid=(r // br,),
        in_specs=[
            pl.BlockSpec((br, d), lambda i: (i, 0)),
            pl.BlockSpec((br, d), lambda i: (i, 0)),
            _full((d, d)),
            _full((d, d)),
            _full((d, d)),
            _full((1, d)),
            _full((d, d)),
            _full((1, d)),
            _full((1, d)),
            _full((1, d)),
        ],
        out_specs=pl.BlockSpec((br, d), lambda i: (i, 0)),
        out_shape=jax.ShapeDtypeStruct((r, d), F32),
    )(h, s, w2, nwa, nwb, nb1, nw2, nb2, lg, lb)


def _partition_sc(src, dst):
    mesh = plsc.VectorSubcoreMesh(core_axis_name="c", subcore_axis_name="s")

    @functools.partial(
        pl.kernel,
        out_type=(
            jax.ShapeDtypeStruct((_NW, 8, _CAP), I32),
            jax.ShapeDtypeStruct((_NW, 8, _CAP), I32),
            jax.ShapeDtypeStruct((_NW, 8, _CAP), I32),
            jax.ShapeDtypeStruct((_NW, 8, 16), I32),
        ),
        mesh=mesh,
        compiler_params=pltpu.CompilerParams(needs_layout_passes=False),
        scratch_types=[
            pltpu.VMEM((_PCH,), I32),
            pltpu.VMEM((_PCH,), I32),
            pltpu.VMEM((_CAP,), I32),
            pltpu.VMEM((_CAP,), I32),
            pltpu.VMEM((_CAP,), I32),
            pltpu.VMEM((16,), I32),
            pltpu.SemaphoreType.DMA,
            pltpu.SemaphoreType.DMA,
        ],
    )
    def body(src_hbm, dst_hbm, osrc, odstl, oeid, ocnt,
             srcv, dstv, bsrc, bdstl, beid, cntb, psem0, psem1):
        cid = lax.axis_index("c")
        sid = lax.axis_index("s")
        w = cid * _NT + sid
        zi = jnp.zeros((16,), I32)
        dumv = jnp.full((16,), _BK, I32)

        def pre(r, carry):
            sl = pl.ds(r * 16, 16)
            bsrc[sl] = zi
            beid[sl] = zi
            bdstl[sl] = dumv
            return carry

        lax.fori_loop(0, _CAP // 16, pre, 0)
        lo = w * _BK
        lanes = lax.iota(I32, 16)
        limit = _CAP - 16

        def chunk(i, off):
            base = i * _PCH
            pc0 = pltpu.async_copy(src_hbm.at[pl.ds(base, _PCH)], srcv, psem0)
            pc1 = pltpu.async_copy(dst_hbm.at[pl.ds(base, _PCH)], dstv, psem1)
            pc0.wait()
            pc1.wait()

            def slc(j, off2):
                sl = pl.ds(j * 16, 16)
                sv = srcv[sl]
                dv = dstv[sl]
                eid = base + j * 16 + lanes
                msk = (dv >= lo) & (dv < lo + _BK)
                pos = plsc.cumsum(msk.astype(I32))
                idx = jnp.minimum(off2 + pos - 1, limit)
                plsc.store_scatter(bsrc, [idx], sv, mask=msk)
                plsc.store_scatter(bdstl, [idx], dv - lo, mask=msk)
                plsc.store_scatter(beid, [idx], eid, mask=msk)
                return jnp.minimum(off2 + jnp.max(pos), limit)

            return lax.fori_loop(0, _PCH // 16, slc, off)

        off = lax.fori_loop(0, _NCHP, chunk, 0)
        nch = (off + _CH - 1) // _CH
        cntb[...] = zi + nch
        pltpu.sync_copy(bsrc, osrc.at[w, 0])
        pltpu.sync_copy(bdstl, odstl.at[w, 0])
        pltpu.sync_copy(beid, oeid.at[w, 0])
        pltpu.sync_copy(cntb, ocnt.at[w, 0])

    return body(src, dst)


def _edge_sc(g, f, c, psrc, pdstl, peid, pcnt):
    mesh = plsc.VectorSubcoreMesh(core_axis_name="c", subcore_axis_name="s")

    @functools.partial(
        pl.kernel,
        out_type=jax.ShapeDtypeStruct((_NPAD, _H), F32),
        mesh=mesh,
        compiler_params=pltpu.CompilerParams(needs_layout_passes=False),
        scratch_types=[
            pltpu.VMEM((16,), I32),
            pltpu.VMEM((_CH,), I32),
            pltpu.VMEM((_CH,), I32),
            pltpu.VMEM((_CH + 16,), I32),
            pltpu.VMEM((_CH,), I32),
            pltpu.VMEM((_CH, _H), F32),
            pltpu.VMEM((_CH, _H), F32),
            pltpu.VMEM((_CH, _H), F32),
            pltpu.VMEM((_BK + 8, _H), F32),
            pltpu.SemaphoreType.DMA,
            pltpu.SemaphoreType.DMA,
            pltpu.SemaphoreType.DMA,
        ],
    )
    def body(g_hbm, f_hbm, c_hbm, psrc_hbm, pdstl_hbm, peid_hbm,
             pcnt_hbm, s_out,
             cntv, srcv, gidv, dstlv, eidv, gv, fv, cv, acc,
             sem0, sem1, sem2):
        cid = lax.axis_index("c")
        sid = lax.axis_index("s")
        w = cid * _NT + sid
        zv = jnp.zeros((16,), F32)

        def initr(r, carry):
            for j in range(_H // 16):
                acc[r, pl.ds(j * 16, 16)] = zv
            return carry

        lax.fori_loop(0, _BK + 8, initr, 0)
        row0 = w * _BK
        pltpu.sync_copy(pcnt_hbm.at[w, 0], cntv)
        nch = cntv[pl.ds(0, 16)][0]

        def chunk(i, carry):
            base = i * _CH
            ci0 = pltpu.async_copy(psrc_hbm.at[w, 0, pl.ds(base, _CH)], srcv, sem0)
            ci1 = pltpu.async_copy(pdstl_hbm.at[w, 0, pl.ds(base, _CH)],
                                   dstlv.at[pl.ds(0, _CH)], sem1)
            ci2 = pltpu.async_copy(peid_hbm.at[w, 0, pl.ds(base, _CH)], eidv, sem2)
            ci0.wait()
            ci1.wait()
            ci2.wait()

            def gidr(j, carry2):
                sl = pl.ds(j * 16, 16)
                gidv[sl] = jnp.minimum(dstlv[sl] + row0, _NPAD - 1)
                return carry2

            lax.fori_loop(0, _CH // 16, gidr, 0)
            cpg = pltpu.async_copy(g_hbm.at[srcv], gv, sem0)
            cpf = pltpu.async_copy(f_hbm.at[gidv], fv, sem1)
            cpc = pltpu.async_copy(c_hbm.at[eidv], cv, sem2)
            cpg.wait()
            cpf.wait()
            cpc.wait()

            def rowb(r2, rc):
                ra = r2 * 2
                rb = ra + 1
                rowa = dstlv[pl.ds(ra, 16)][0]
                rowb_ = dstlv[pl.ds(rb, 16)][0]
                for j in range(_H // 16):
                    sl = pl.ds(j * 16, 16)
                    hida = jnp.maximum(gv[ra, sl] + fv[ra, sl] + cv[ra, sl], 0.0)
                    hidb = jnp.maximum(gv[rb, sl] + fv[rb, sl] + cv[rb, sl], 0.0)
                    plsc.addupdate(acc.at[rowa, sl], hida)
                    plsc.addupdate(acc.at[rowb_, sl], hidb)
                return rc

            lax.fori_loop(0, _CH // 2, rowb, 0)
            return carry

        lax.fori_loop(0, nch, chunk, 0)
        for r0, nr in ((0, 160), (160, 160)):
            pltpu.sync_copy(acc.at[pl.ds(r0, nr)], s_out.at[pl.ds(row0 + r0, nr)])

    return body(g, f, c, psrc, pdstl, peid, pcnt)


def kernel(x, edge_index, edge_attr, params):
    n, d = x.shape
    e = edge_index.shape[1]

    src = edge_index[0]
    dst = edge_index[1]
    src_p = jnp.concatenate([src, jnp.zeros((_EPAD - e,), I32)])
    dst_p = jnp.concatenate([dst, jnp.full((_EPAD - e,), n, I32)])
    ea_p = jnp.concatenate([edge_attr, jnp.zeros((_EPAD - e, edge_attr.shape[1]), F32)])
    x_p = jnp.concatenate([x, jnp.zeros((_NPAD - n, d), F32)])

    def r2(b):
        return b.reshape(1, -1)

    enc = params['enc']
    h = _tc_mlp(x_p, enc['W1'], r2(enc['b1']), enc['W2'], r2(enc['b2']), br=2048)
    psrc, pdstl, peid, pcnt = _partition_sc(src_p, dst_p)

    for lp in params['layers']:
        ew1 = lp['edge']['W1']
        w1a, w1b, w1c = ew1[:_H], ew1[_H:2 * _H], ew1[2 * _H:]
        c = _tc_lin(ea_p, w1c, r2(lp['edge']['b1']), br=2048)
        g, f = _tc_pre(h, w1a, w1b, br=2048)
        s = _edge_sc(g, f, c, psrc, pdstl, peid, pcnt)
        nw1 = lp['node']['W1']
        h = _tc_post(h, s, lp['edge']['W2'],
                     nw1[:_H], nw1[_H:], r2(lp['node']['b1']),
                     lp['node']['W2'], r2(lp['node']['b2']),
                     r2(lp['ln']['g']), r2(lp['ln']['b']), br=2048)

    hd = params['head']
    hw2 = jnp.zeros((_H, _H), F32).at[:, :1].set(hd['W2'])
    hb2 = jnp.zeros((1, _H), F32).at[0, 0].set(hd['b2'][0])
    out = _tc_mlp(h, hd['W1'], r2(hd['b1']), hw2, hb2, br=2048)
    return out[:n, :1]
